# baseline (speedup 1.0000x reference)
import functools
import jax
import jax.numpy as jnp
from jax import lax
from jax.experimental import pallas as pl
from jax.experimental.pallas import tpu as pltpu
from jax.experimental.pallas import tpu_sc as plsc
from jax.experimental import layout as jax_layout

NUM_CLASSES = 1000
C, H, W = 3, 32, 32
D = C * H * W
BATCH = 16384
NC, NS = 2, 16
NW = NC * NS
NSPLIT = 2
ROWS = BATCH // NSPLIT
BPW = ROWS // NW
K = 4
NCHUNK = BPW // K
NBUF = 2
LANES = 16
COLS = D // LANES


def _sc_body(means_hbm, stds_hbm, labels_hbm, noise_hbm, out_hbm,
             idx_v, mean_v, std_v, noise_v, out_v, sem_in, sem_out):
    wid = lax.axis_index("s") * NC + lax.axis_index("c")
    base = wid * BPW

    pltpu.sync_copy(labels_hbm.at[wid], idx_v)

    def start_in(b, c):
        row0 = base + c * K
        pltpu.async_copy(means_hbm.at[idx_v.at[c]], mean_v[b], sem_in[b])
        pltpu.async_copy(stds_hbm.at[idx_v.at[c]], std_v[b], sem_in[b])
        pltpu.async_copy(noise_hbm.at[pl.ds(row0, K)], noise_v[b], sem_in[b])

    def wait_in(b):
        pltpu.make_async_copy(means_hbm.at[idx_v.at[0]], mean_v[b],
                              sem_in[b]).wait()
        pltpu.make_async_copy(stds_hbm.at[idx_v.at[0]], std_v[b],
                              sem_in[b]).wait()
        pltpu.make_async_copy(noise_hbm.at[pl.ds(base, K)], noise_v[b],
                              sem_in[b]).wait()

    def start_out(b, c):
        row0 = base + c * K
        pltpu.async_copy(out_v[b], out_hbm.at[pl.ds(row0, K)], sem_out[b])

    def wait_out(b):
        pltpu.make_async_copy(out_v[b], out_hbm.at[pl.ds(base, K)],
                              sem_out[b]).wait()

    for b in range(NBUF):
        start_in(b, b)

    def iteration(i, carry):
        for b in range(NBUF):
            cc = i * NBUF + b
            wait_in(b)

            @pl.when(cc >= NBUF)
            def _():
                wait_out(b)

            def col(j, carry2):
                off = j * LANES
                for k in range(K):
                    n = noise_v[b][k, pl.ds(off, LANES)]
                    m = mean_v[b][k, pl.ds(off, LANES)]
                    s = std_v[b][k, pl.ds(off, LANES)]
                    out_v[b][k, pl.ds(off, LANES)] = m + s * n
                return carry2

            lax.fori_loop(0, COLS, col, 0)
            start_out(b, cc)

            @pl.when(cc + NBUF < NCHUNK)
            def _():
                start_in(b, cc + NBUF)
        return carry

    lax.fori_loop(0, NCHUNK // NBUF, iteration, 0)
    for b in range(NBUF):
        wait_out(b)


@functools.partial(
    pl.kernel,
    out_type=jax.ShapeDtypeStruct((ROWS, D), jnp.float32),
    mesh=plsc.VectorSubcoreMesh(
        core_axis_name="c", subcore_axis_name="s",
        num_cores=NC, num_subcores=NS),
    scratch_types=[
        pltpu.VMEM((NCHUNK, K), jnp.int32),
        [pltpu.VMEM((K, D), jnp.float32) for _ in range(NBUF)],
        [pltpu.VMEM((K, D), jnp.float32) for _ in range(NBUF)],
        [pltpu.VMEM((K, D), jnp.float32) for _ in range(NBUF)],
        [pltpu.VMEM((K, D), jnp.float32) for _ in range(NBUF)],
        [pltpu.SemaphoreType.DMA for _ in range(NBUF)],
        [pltpu.SemaphoreType.DMA for _ in range(NBUF)],
    ],
)
def _sc_embed(means_hbm, stds_hbm, labels_hbm, noise_hbm, out_hbm,
              idx_v, mean_v, std_v, noise_v, out_v, sem_in, sem_out):
    _sc_body(means_hbm, stds_hbm, labels_hbm, noise_hbm, out_hbm,
             idx_v, mean_v, std_v, noise_v, out_v, sem_in, sem_out)


def _kernel_impl(labels, noise, class_means, class_stds):
    means2 = class_means.reshape(NUM_CLASSES, D)
    stds2 = class_stds.reshape(NUM_CLASSES, D)
    outs = []
    for i in range(NSPLIT):
        lab = lax.slice_in_dim(labels, i * ROWS, (i + 1) * ROWS)
        nz = lax.slice_in_dim(noise, i * ROWS, (i + 1) * ROWS)
        out = _sc_embed(means2, stds2,
                        lab.reshape(NW, NCHUNK, K),
                        nz.reshape(ROWS, D))
        outs.append(out.reshape(ROWS, C, H, W))
    if NSPLIT == 1:
        return outs[0]
    return jnp.concatenate(outs, axis=0)


kernel = jax.jit(_kernel_impl)


# speedup vs baseline: 1.1077x; 1.1077x over previous
import functools
import jax
import jax.numpy as jnp
from jax import lax
from jax.experimental import pallas as pl
from jax.experimental.pallas import tpu as pltpu
from jax.experimental.pallas import tpu_sc as plsc
from jax.experimental import layout as jax_layout

NUM_CLASSES = 1000
C, H, W = 3, 32, 32
D = C * H * W
BATCH = 16384
NC, NS = 2, 16
NW = NC * NS
NSPLIT = 1
ROWS = BATCH // NSPLIT
BPW = ROWS // NW
K = 4
NCHUNK = BPW // K
NBUF = 2
LANES = 16
COLS = D // LANES
UNROLL = 2


def _sc_body(means_hbm, stds_hbm, labels_hbm, noise_hbm, out_hbm,
             idx_v, mean_v, std_v, noise_v, out_v, sem_in, sem_out):
    wid = lax.axis_index("s") * NC + lax.axis_index("c")
    base = wid * BPW

    pltpu.sync_copy(labels_hbm.at[wid], idx_v)

    def start_in(b, c):
        row0 = base + c * K
        pltpu.async_copy(means_hbm.at[idx_v.at[c]], mean_v[b], sem_in[b])
        pltpu.async_copy(stds_hbm.at[idx_v.at[c]], std_v[b], sem_in[b])
        pltpu.async_copy(noise_hbm.at[pl.ds(row0, K)], noise_v[b], sem_in[b])

    def wait_in(b):
        pltpu.make_async_copy(means_hbm.at[idx_v.at[0]], mean_v[b],
                              sem_in[b]).wait()
        pltpu.make_async_copy(stds_hbm.at[idx_v.at[0]], std_v[b],
                              sem_in[b]).wait()
        pltpu.make_async_copy(noise_hbm.at[pl.ds(base, K)], noise_v[b],
                              sem_in[b]).wait()

    def start_out(b, c):
        row0 = base + c * K
        pltpu.async_copy(out_v[b], out_hbm.at[pl.ds(row0, K)], sem_out[b])

    def wait_out(b):
        pltpu.make_async_copy(out_v[b], out_hbm.at[pl.ds(base, K)],
                              sem_out[b]).wait()

    for b in range(NBUF):
        start_in(b, b)

    def iteration(i, carry):
        for b in range(NBUF):
            cc = i * NBUF + b
            wait_in(b)

            @pl.when(cc >= NBUF)
            def _():
                wait_out(b)

            def col(j, carry2):
                off0 = j * (LANES * UNROLL)
                for u in range(UNROLL):
                    off = off0 + u * LANES
                    for k in range(K):
                        n = noise_v[b][k, pl.ds(off, LANES)]
                        m = mean_v[b][k, pl.ds(off, LANES)]
                        s = std_v[b][k, pl.ds(off, LANES)]
                        out_v[b][k, pl.ds(off, LANES)] = m + s * n
                return carry2

            lax.fori_loop(0, COLS // UNROLL, col, 0)
            start_out(b, cc)

            @pl.when(cc + NBUF < NCHUNK)
            def _():
                start_in(b, cc + NBUF)
        return carry

    lax.fori_loop(0, NCHUNK // NBUF, iteration, 0)
    for b in range(NBUF):
        wait_out(b)


@functools.partial(
    pl.kernel,
    out_type=jax.ShapeDtypeStruct((ROWS, D), jnp.float32),
    mesh=plsc.VectorSubcoreMesh(
        core_axis_name="c", subcore_axis_name="s",
        num_cores=NC, num_subcores=NS),
    scratch_types=[
        pltpu.VMEM((NCHUNK, K), jnp.int32),
        [pltpu.VMEM((K, D), jnp.float32) for _ in range(NBUF)],
        [pltpu.VMEM((K, D), jnp.float32) for _ in range(NBUF)],
        [pltpu.VMEM((K, D), jnp.float32) for _ in range(NBUF)],
        [pltpu.VMEM((K, D), jnp.float32) for _ in range(NBUF)],
        [pltpu.SemaphoreType.DMA for _ in range(NBUF)],
        [pltpu.SemaphoreType.DMA for _ in range(NBUF)],
    ],
)
def _sc_embed(means_hbm, stds_hbm, labels_hbm, noise_hbm, out_hbm,
              idx_v, mean_v, std_v, noise_v, out_v, sem_in, sem_out):
    _sc_body(means_hbm, stds_hbm, labels_hbm, noise_hbm, out_hbm,
             idx_v, mean_v, std_v, noise_v, out_v, sem_in, sem_out)


def _kernel_impl(labels, noise, class_means, class_stds):
    means2 = class_means.reshape(NUM_CLASSES, D)
    stds2 = class_stds.reshape(NUM_CLASSES, D)
    outs = []
    for i in range(NSPLIT):
        lab = lax.slice_in_dim(labels, i * ROWS, (i + 1) * ROWS)
        nz = lax.slice_in_dim(noise, i * ROWS, (i + 1) * ROWS)
        out = _sc_embed(means2, stds2,
                        lab.reshape(NW, NCHUNK, K),
                        nz.reshape(ROWS, D))
        outs.append(out.reshape(ROWS, C, H, W))
    if NSPLIT == 1:
        return outs[0]
    return jnp.concatenate(outs, axis=0)


kernel = jax.jit(_kernel_impl)


# speedup vs baseline: 1.3181x; 1.1899x over previous
import functools
import jax
import jax.numpy as jnp
from jax import lax
from jax.experimental import pallas as pl
from jax.experimental.pallas import tpu as pltpu
from jax.experimental.pallas import tpu_sc as plsc
from jax.experimental import layout as jax_layout

NUM_CLASSES = 1000
C, H, W = 3, 32, 32
D = C * H * W
BATCH = 16384
NC, NS = 2, 16
NW = NC * NS
NSPLIT = 1
ROWS = BATCH // NSPLIT
BPW = ROWS // NW
K = 2
NCHUNK = BPW // K
NBUF = 4
LANES = 16
COLS = D // LANES
UNROLL = 1


def _sc_body(means_hbm, stds_hbm, labels_hbm, noise_hbm, out_hbm,
             idx_v, mean_v, std_v, noise_v, out_v, sem_in, sem_out):
    wid = lax.axis_index("s") * NC + lax.axis_index("c")
    base = wid * BPW

    pltpu.sync_copy(labels_hbm.at[wid], idx_v)

    def start_in(b, c):
        row0 = base + c * K
        pltpu.async_copy(means_hbm.at[idx_v.at[c]], mean_v[b], sem_in[b])
        pltpu.async_copy(stds_hbm.at[idx_v.at[c]], std_v[b], sem_in[b])
        pltpu.async_copy(noise_hbm.at[pl.ds(row0, K)], noise_v[b], sem_in[b])

    def wait_in(b):
        pltpu.make_async_copy(means_hbm.at[idx_v.at[0]], mean_v[b],
                              sem_in[b]).wait()
        pltpu.make_async_copy(stds_hbm.at[idx_v.at[0]], std_v[b],
                              sem_in[b]).wait()
        pltpu.make_async_copy(noise_hbm.at[pl.ds(base, K)], noise_v[b],
                              sem_in[b]).wait()

    def start_out(b, c):
        row0 = base + c * K
        pltpu.async_copy(out_v[b], out_hbm.at[pl.ds(row0, K)], sem_out[b])

    def wait_out(b):
        pltpu.make_async_copy(out_v[b], out_hbm.at[pl.ds(base, K)],
                              sem_out[b]).wait()

    for b in range(NBUF):
        start_in(b, b)

    def iteration(i, carry):
        for b in range(NBUF):
            cc = i * NBUF + b
            wait_in(b)

            @pl.when(cc >= NBUF)
            def _():
                wait_out(b)

            def col(j, carry2):
                off0 = j * (LANES * UNROLL)
                for u in range(UNROLL):
                    off = off0 + u * LANES
                    for k in range(K):
                        n = noise_v[b][k, pl.ds(off, LANES)]
                        m = mean_v[b][k, pl.ds(off, LANES)]
                        s = std_v[b][k, pl.ds(off, LANES)]
                        out_v[b][k, pl.ds(off, LANES)] = m + s * n
                return carry2

            lax.fori_loop(0, COLS // UNROLL, col, 0)
            start_out(b, cc)

            @pl.when(cc + NBUF < NCHUNK)
            def _():
                start_in(b, cc + NBUF)
        return carry

    lax.fori_loop(0, NCHUNK // NBUF, iteration, 0)
    for b in range(NBUF):
        wait_out(b)


@functools.partial(
    pl.kernel,
    out_type=jax.ShapeDtypeStruct((ROWS, D), jnp.float32),
    mesh=plsc.VectorSubcoreMesh(
        core_axis_name="c", subcore_axis_name="s",
        num_cores=NC, num_subcores=NS),
    scratch_types=[
        pltpu.VMEM((NCHUNK, K), jnp.int32),
        [pltpu.VMEM((K, D), jnp.float32) for _ in range(NBUF)],
        [pltpu.VMEM((K, D), jnp.float32) for _ in range(NBUF)],
        [pltpu.VMEM((K, D), jnp.float32) for _ in range(NBUF)],
        [pltpu.VMEM((K, D), jnp.float32) for _ in range(NBUF)],
        [pltpu.SemaphoreType.DMA for _ in range(NBUF)],
        [pltpu.SemaphoreType.DMA for _ in range(NBUF)],
    ],
)
def _sc_embed(means_hbm, stds_hbm, labels_hbm, noise_hbm, out_hbm,
              idx_v, mean_v, std_v, noise_v, out_v, sem_in, sem_out):
    _sc_body(means_hbm, stds_hbm, labels_hbm, noise_hbm, out_hbm,
             idx_v, mean_v, std_v, noise_v, out_v, sem_in, sem_out)


def _kernel_impl(labels, noise, class_means, class_stds):
    means2 = class_means.reshape(NUM_CLASSES, D)
    stds2 = class_stds.reshape(NUM_CLASSES, D)
    outs = []
    for i in range(NSPLIT):
        lab = lax.slice_in_dim(labels, i * ROWS, (i + 1) * ROWS)
        nz = lax.slice_in_dim(noise, i * ROWS, (i + 1) * ROWS)
        out = _sc_embed(means2, stds2,
                        lab.reshape(NW, NCHUNK, K),
                        nz.reshape(ROWS, D))
        outs.append(out.reshape(ROWS, C, H, W))
    if NSPLIT == 1:
        return outs[0]
    return jnp.concatenate(outs, axis=0)


kernel = jax.jit(_kernel_impl)


# speedup vs baseline: 1.3183x; 1.0002x over previous
import functools
import jax
import jax.numpy as jnp
from jax import lax
from jax.experimental import pallas as pl
from jax.experimental.pallas import tpu as pltpu
from jax.experimental.pallas import tpu_sc as plsc
from jax.experimental import layout as jax_layout

NUM_CLASSES = 1000
C, H, W = 3, 32, 32
D = C * H * W
BATCH = 16384
NC, NS = 2, 16
NW = NC * NS
NSPLIT = 1
ROWS = BATCH // NSPLIT
BPW = ROWS // NW
K = 2
NCHUNK = BPW // K
NBUF = 4
LANES = 16
COLS = D // LANES
UNROLL = 1


def _sc_body(means_hbm, stds_hbm, labels_hbm, noise_hbm, out_hbm,
             idx_v, mean_v, std_v, noise_v, out_v, sem_in, sem_out):
    wid = lax.axis_index("s") * NC + lax.axis_index("c")
    base = wid * BPW

    pltpu.sync_copy(labels_hbm.at[wid], idx_v)

    def start_in(b, c):
        row0 = base + c * K
        pltpu.async_copy(means_hbm.at[idx_v.at[c]], mean_v[b], sem_in[b])
        pltpu.async_copy(stds_hbm.at[idx_v.at[c]], std_v[b], sem_in[b])
        pltpu.async_copy(noise_hbm.at[pl.ds(row0, K)], noise_v[b], sem_in[b])

    def wait_in(b):
        pltpu.make_async_copy(means_hbm.at[idx_v.at[0]], mean_v[b],
                              sem_in[b]).wait()
        pltpu.make_async_copy(stds_hbm.at[idx_v.at[0]], std_v[b],
                              sem_in[b]).wait()
        pltpu.make_async_copy(noise_hbm.at[pl.ds(base, K)], noise_v[b],
                              sem_in[b]).wait()

    def start_out(b, c):
        row0 = base + c * K
        pltpu.async_copy(out_v[b], out_hbm.at[pl.ds(row0, K)], sem_out[b])

    def wait_out(b):
        pltpu.make_async_copy(out_v[b], out_hbm.at[pl.ds(base, K)],
                              sem_out[b]).wait()

    for b in range(NBUF):
        start_in(b, b)

    def iteration(i, carry):
        for b in range(NBUF):
            cc = i * NBUF + b
            wait_in(b)

            @pl.when(cc >= NBUF)
            def _():
                wait_out(b)

            def col(j, carry2):
                off0 = j * (LANES * UNROLL)
                for u in range(UNROLL):
                    off = off0 + u * LANES
                    for k in range(K):
                        n = noise_v[b][k, pl.ds(off, LANES)]
                        m = mean_v[b][k, pl.ds(off, LANES)]
                        s = std_v[b][k, pl.ds(off, LANES)]
                        out_v[b][k, pl.ds(off, LANES)] = m + s * n
                return carry2

            lax.fori_loop(0, COLS // UNROLL, col, 0)
            start_out(b, cc)

            @pl.when(cc + NBUF < NCHUNK)
            def _():
                start_in(b, cc + NBUF)
        return carry

    lax.fori_loop(0, NCHUNK // NBUF, iteration, 0)
    for b in range(NBUF):
        wait_out(b)


@functools.partial(
    pl.kernel,
    out_type=jax.ShapeDtypeStruct((ROWS, D), jnp.float32),
    mesh=plsc.VectorSubcoreMesh(
        core_axis_name="c", subcore_axis_name="s",
        num_cores=NC, num_subcores=NS),
    compiler_params=pltpu.CompilerParams(
        allow_input_fusion=[False, False, False, True]),
    scratch_types=[
        pltpu.VMEM((NCHUNK, K), jnp.int32),
        [pltpu.VMEM((K, D), jnp.float32) for _ in range(NBUF)],
        [pltpu.VMEM((K, D), jnp.float32) for _ in range(NBUF)],
        [pltpu.VMEM((K, D), jnp.float32) for _ in range(NBUF)],
        [pltpu.VMEM((K, D), jnp.float32) for _ in range(NBUF)],
        [pltpu.SemaphoreType.DMA for _ in range(NBUF)],
        [pltpu.SemaphoreType.DMA for _ in range(NBUF)],
    ],
)
def _sc_embed(means_hbm, stds_hbm, labels_hbm, noise_hbm, out_hbm,
              idx_v, mean_v, std_v, noise_v, out_v, sem_in, sem_out):
    _sc_body(means_hbm, stds_hbm, labels_hbm, noise_hbm, out_hbm,
             idx_v, mean_v, std_v, noise_v, out_v, sem_in, sem_out)


def _kernel_impl(labels, noise, class_means, class_stds):
    means2 = class_means.reshape(NUM_CLASSES, D)
    stds2 = class_stds.reshape(NUM_CLASSES, D)
    outs = []
    for i in range(NSPLIT):
        lab = lax.slice_in_dim(labels, i * ROWS, (i + 1) * ROWS)
        nz = lax.slice_in_dim(noise, i * ROWS, (i + 1) * ROWS)
        out = _sc_embed(means2, stds2,
                        lab.reshape(NW, NCHUNK, K),
                        nz.reshape(ROWS, D))
        outs.append(out.reshape(ROWS, C, H, W))
    if NSPLIT == 1:
        return outs[0]
    return jnp.concatenate(outs, axis=0)


kernel = jax.jit(_kernel_impl)
